# Initial kernel scaffold; baseline (speedup 1.0000x reference)
import functools

import jax
import jax.numpy as jnp
from jax import lax
from jax.experimental import pallas as pl
from jax.experimental.pallas import tpu as pltpu
from jax.experimental.pallas import tpu_sc as plsc

B = 16384 * 200
D = 32
NC, NS = 2, 16
NW = NC * NS
BPW = B // NW
C = 2048
NCH = BPW // C

_mesh = plsc.VectorSubcoreMesh(core_axis_name="c", subcore_axis_name="s")


@functools.partial(
    pl.kernel,
    out_type=jax.ShapeDtypeStruct((B, D), jnp.float32),
    mesh=_mesh,
    scratch_types=[
        pltpu.VMEM((C,), jnp.int32),
        pltpu.VMEM((C, D), jnp.float32),
        pltpu.SemaphoreType.DMA,
    ],
)
def _gather(idx_hbm, table_hbm, out_hbm, idx_v, rows_v, sem):
    wid = lax.axis_index("s") * NC + lax.axis_index("c")
    base = wid * BPW

    def body(i, carry):
        off = base + i * C
        pltpu.sync_copy(idx_hbm.at[pl.ds(off, C)], idx_v)
        pltpu.async_copy(table_hbm.at[idx_v], rows_v, sem).wait()
        pltpu.sync_copy(rows_v, out_hbm.at[pl.ds(off, C)])
        return carry

    lax.fori_loop(0, NCH, body, 0)


def kernel(round_numbers, embedding_table):
    idx = round_numbers.reshape(-1).astype(jnp.int32)
    out = _gather(idx, embedding_table)
    return out.reshape(round_numbers.shape + (D,))


# speedup vs baseline: 6.3229x; 6.3229x over previous
import functools

import jax
import jax.numpy as jnp
from jax import lax
from jax.experimental import pallas as pl
from jax.experimental.pallas import tpu as pltpu
from jax.experimental.pallas import tpu_sc as plsc

B = 16384 * 200
D = 32
NC, NS = 2, 16
NW = NC * NS
BPW = B // NW
C = 2048
NCH = BPW // C

_mesh = plsc.VectorSubcoreMesh(core_axis_name="c", subcore_axis_name="s")


@functools.partial(
    pl.kernel,
    out_type=jax.ShapeDtypeStruct((B, D), jnp.float32),
    mesh=_mesh,
    scratch_types=[
        pltpu.VMEM((C,), jnp.int32),
        pltpu.VMEM((C, D), jnp.float32),
        pltpu.SemaphoreType.DMA,
    ],
    compiler_params=pltpu.CompilerParams(use_tc_tiling_on_sc=False),
)
def _gather(idx_hbm, table_hbm, out_hbm, idx_v, rows_v, sem):
    wid = lax.axis_index("s") * NC + lax.axis_index("c")
    base = wid * BPW

    def body(i, carry):
        off = base + i * C
        pltpu.sync_copy(idx_hbm.at[pl.ds(off, C)], idx_v)
        pltpu.async_copy(table_hbm.at[idx_v], rows_v, sem).wait()
        pltpu.sync_copy(rows_v, out_hbm.at[pl.ds(off, C)])
        return carry

    lax.fori_loop(0, NCH, body, 0)


def kernel(round_numbers, embedding_table):
    idx = round_numbers.reshape(-1).astype(jnp.int32)
    out = _gather(idx, embedding_table)
    return out.reshape(round_numbers.shape + (D,))


# speedup vs baseline: 6.4864x; 1.0259x over previous
import functools

import jax
import jax.numpy as jnp
from jax import lax
from jax.experimental import pallas as pl
from jax.experimental.pallas import tpu as pltpu
from jax.experimental.pallas import tpu_sc as plsc

B = 16384 * 200
D = 32
NC, NS = 2, 16
NW = NC * NS
BPW = B // NW
C = 1600
NCH = BPW // C
NBUF = 2
NGRP = NCH // NBUF

_mesh = plsc.VectorSubcoreMesh(core_axis_name="c", subcore_axis_name="s")


@functools.partial(
    pl.kernel,
    out_type=jax.ShapeDtypeStruct((B, D), jnp.float32),
    mesh=_mesh,
    scratch_types=[
        pltpu.VMEM((NBUF, C), jnp.int32),
        pltpu.VMEM((NBUF, C, D), jnp.float32),
        [pltpu.SemaphoreType.DMA] * NBUF,
        [pltpu.SemaphoreType.DMA] * NBUF,
    ],
    compiler_params=pltpu.CompilerParams(use_tc_tiling_on_sc=False),
)
def _gather(idx_hbm, table_hbm, out_hbm, idx_v, rows_v, gsems, wsems):
    wid = lax.axis_index("s") * NC + lax.axis_index("c")
    base = wid * BPW

    def group(g, carry):
        base_g = base + g * (NBUF * C)
        for b in range(NBUF):
            off = base_g + b * C

            @pl.when(g > 0)
            def _drain(b=b, off=off):
                pltpu.make_async_copy(
                    rows_v.at[b], out_hbm.at[pl.ds(off - NBUF * C, C)], wsems[b]
                ).wait()

            pltpu.sync_copy(idx_hbm.at[pl.ds(off, C)], idx_v.at[b])
            pltpu.async_copy(table_hbm.at[idx_v.at[b]], rows_v.at[b], gsems[b])
        for b in range(NBUF):
            off = base_g + b * C
            pltpu.make_async_copy(
                table_hbm.at[idx_v.at[b]], rows_v.at[b], gsems[b]
            ).wait()
            pltpu.async_copy(rows_v.at[b], out_hbm.at[pl.ds(off, C)], wsems[b])
        return carry

    lax.fori_loop(0, NGRP, group, 0)
    for b in range(NBUF):
        off = base + (NGRP - 1) * (NBUF * C) + b * C
        pltpu.make_async_copy(
            rows_v.at[b], out_hbm.at[pl.ds(off, C)], wsems[b]
        ).wait()


def kernel(round_numbers, embedding_table):
    idx = round_numbers.reshape(-1).astype(jnp.int32)
    out = _gather(idx, embedding_table)
    return out.reshape(round_numbers.shape + (D,))
